# baseline (speedup 1.0000x reference)
import functools

import jax
import jax.numpy as jnp
from jax import lax
from jax.experimental import pallas as pl
from jax.experimental.pallas import tpu as pltpu
from jax.experimental.pallas import tpu_sc as plsc

EMB_DIM = 32
NUM_EMB = 8192
ROW_TILE = 640


def _make_vq_tc_body(hw):
    def _vq_tc_body(z_ref, cba_ref, idx_ref, loss_ref):
        b = pl.program_id(0)
        j = pl.program_id(1)
        zt = z_ref[0]
        cb2 = cba_ref[...]
        L = zt.shape[1]
        cbnorm = 0.25 * jnp.sum(cb2 * cb2, axis=1, keepdims=True)
        scores2 = jnp.dot(cb2, zt, preferred_element_type=jnp.float32)
        d = cbnorm - scores2
        idx_ref[0, 0, 0, :] = jnp.argmin(d, axis=0).astype(jnp.int32)
        dmin = (jnp.min(d, axis=0, keepdims=True)
                + jnp.sum(zt * zt, axis=0, keepdims=True))
        lane = jax.lax.broadcasted_iota(jnp.int32, (1, L), 1) + j * L
        dmin = jnp.where(lane < hw, dmin, 0.0)

        @pl.when((b == 0) & (j == 0))
        def _():
            loss_ref[...] = jnp.zeros((1, 1), jnp.float32)
        loss_ref[...] += jnp.sum(dmin).reshape(1, 1)
    return _vq_tc_body


def _vq_argmin(z3, cb_aug, hw):
    B, C, HWp = z3.shape
    grid = (B, HWp // ROW_TILE)
    idx_out, loss_sum = pl.pallas_call(
        _make_vq_tc_body(hw),
        grid=grid,
        in_specs=[
            pl.BlockSpec((1, C, ROW_TILE), lambda b, j: (b, 0, j)),
            pl.BlockSpec((NUM_EMB, EMB_DIM), lambda b, j: (0, 0)),
        ],
        out_specs=[
            pl.BlockSpec((1, 1, 1, ROW_TILE), lambda b, j: (b, j, 0, 0)),
            pl.BlockSpec((1, 1), lambda b, j: (0, 0)),
        ],
        out_shape=[
            jax.ShapeDtypeStruct((B, HWp // ROW_TILE, 1, ROW_TILE), jnp.int32),
            jax.ShapeDtypeStruct((1, 1), jnp.float32),
        ],
    )(z3, cb_aug)
    return idx_out.reshape(B * HWp), loss_sum[0, 0]


_SC_CORES = 2
_SC_SUBCORES = 16
_NW = _SC_CORES * _SC_SUBCORES


def _make_sc_gather(b_pad):
    b_per_w = b_pad // _NW
    mesh = plsc.VectorSubcoreMesh(core_axis_name="c", subcore_axis_name="s")

    @functools.partial(
        pl.kernel, mesh=mesh,
        compiler_params=pltpu.CompilerParams(use_tc_tiling_on_sc=False),
        out_type=jax.ShapeDtypeStruct((b_pad, EMB_DIM), jnp.float32),
        scratch_types=[
            pltpu.VMEM((b_per_w,), jnp.int32),
            pltpu.VMEM((b_per_w, EMB_DIM), jnp.float32),
            pltpu.SemaphoreType.DMA,
        ],
    )
    def sc_gather(table_hbm, idx_hbm, out_hbm, idx_v, rows_v, sem):
        wid = lax.axis_index("s") * _SC_CORES + lax.axis_index("c")
        base = wid * b_per_w
        pltpu.sync_copy(idx_hbm.at[pl.ds(base, b_per_w)], idx_v)
        pltpu.async_copy(table_hbm.at[idx_v], rows_v, sem).wait()
        pltpu.sync_copy(rows_v, out_hbm.at[pl.ds(base, b_per_w)])

    return sc_gather


def _conv(x, w, b, stride=1, pad=1):
    o = lax.conv_general_dilated(x, w, (stride, stride), [(pad, pad), (pad, pad)],
                                 dimension_numbers=('NCHW', 'OIHW', 'NCHW'))
    return o + b[None, :, None, None]


def _convT(x, w, b, stride=2, pad=1):
    k = w.shape[2]
    wf = jnp.flip(w, axis=(2, 3)).transpose(1, 0, 2, 3)
    p = k - 1 - pad
    o = lax.conv_general_dilated(x, wf, (1, 1), [(p, p), (p, p)],
                                 lhs_dilation=(stride, stride),
                                 dimension_numbers=('NCHW', 'OIHW', 'NCHW'))
    return o + b[None, :, None, None]


def _res(x, p, name):
    h = jax.nn.relu(_conv(x, p[name + '_w1'], p[name + '_b1']))
    h = _conv(h, p[name + '_w2'], p[name + '_b2'])
    return jax.nn.relu(x + h)


def _encode(x, p):
    x = jax.nn.relu(_conv(x, p['e_d0_w'], p['e_d0_b'], stride=2, pad=1))
    x = _res(x, p, 'e_s0_r0'); x = _res(x, p, 'e_s0_r1')
    x = jax.nn.relu(_conv(x, p['e_d1_w'], p['e_d1_b'], stride=2, pad=1))
    x = _res(x, p, 'e_s1_r0'); x = _res(x, p, 'e_s1_r1')
    return _conv(x, p['e_out_w'], p['e_out_b'])


def _decode(x, p):
    x = _conv(x, p['d_in_w'], p['d_in_b'])
    x = _res(x, p, 'd_s0_r0'); x = _res(x, p, 'd_s0_r1')
    x = jax.nn.relu(_convT(x, p['d_u0_w'], p['d_u0_b']))
    x = _res(x, p, 'd_s1_r0'); x = _res(x, p, 'd_s1_r1')
    return _convT(x, p['d_u1_w'], p['d_u1_b'])


def kernel(x, params):
    z = _encode(x, params)
    B, C, H, W = z.shape
    n = B * H * W
    hw = H * W
    cb = params['codebook']

    eye = jnp.eye(C, dtype=jnp.float32).reshape(C, C, 1, 1)
    z2 = lax.conv_general_dilated(z, eye, (1, 1), [(0, 0), (0, 0)],
                                  dimension_numbers=('NCHW', 'OIHW', 'NCHW'),
                                  precision=jax.lax.Precision.HIGHEST)

    hw_pad = -(-hw // ROW_TILE) * ROW_TILE
    z3 = jnp.pad(z2.reshape(B, C, hw), ((0, 0), (0, 0), (0, hw_pad - hw)))
    idx_pad, loss_sum = _vq_argmin(z3, cb * 2.0, hw)
    loss = 0.25 * loss_sum / (n * C)

    q_pad = _make_sc_gather(B * hw_pad)(cb, idx_pad)
    q = q_pad.reshape(B, hw_pad, C)[:, :hw, :]

    q_img = q.reshape(B, H, W, C).transpose(0, 3, 1, 2)
    q_st = z + (q_img - z)
    recon = _decode(q_st, params)
    return recon, loss


# speedup vs baseline: 1.1036x; 1.1036x over previous
import functools

import jax
import jax.numpy as jnp
from jax import lax
from jax.experimental import pallas as pl
from jax.experimental.pallas import tpu as pltpu
from jax.experimental.pallas import tpu_sc as plsc

EMB_DIM = 32
NUM_EMB = 8192
ROW_TILE = 640


def _make_vq_tc_body(hw):
    def _vq_tc_body(z_ref, cba_ref, idx_ref, loss_ref):
        b = pl.program_id(0)
        j = pl.program_id(1)
        zt = z_ref[0]
        cb2 = cba_ref[...]
        L = zt.shape[1]
        cbnorm = 0.25 * jnp.sum(cb2 * cb2, axis=1, keepdims=True)
        scores2 = jnp.dot(cb2, zt, preferred_element_type=jnp.float32)
        d = cbnorm - scores2
        idx_ref[0, 0, 0, :] = jnp.argmin(d, axis=0).astype(jnp.int32)
        dmin = (jnp.min(d, axis=0, keepdims=True)
                + jnp.sum(zt * zt, axis=0, keepdims=True))
        lane = jax.lax.broadcasted_iota(jnp.int32, (1, L), 1) + j * L
        dmin = jnp.where(lane < hw, dmin, 0.0)

        @pl.when((b == 0) & (j == 0))
        def _():
            loss_ref[...] = jnp.zeros((1, 1), jnp.float32)
        loss_ref[...] += jnp.sum(dmin).reshape(1, 1)
    return _vq_tc_body


def _vq_argmin(z3, cb_aug, hw):
    B, C, HWp = z3.shape
    grid = (B, HWp // ROW_TILE)
    idx_out, loss_sum = pl.pallas_call(
        _make_vq_tc_body(hw),
        grid=grid,
        in_specs=[
            pl.BlockSpec((1, C, ROW_TILE), lambda b, j: (b, 0, j)),
            pl.BlockSpec((NUM_EMB, EMB_DIM), lambda b, j: (0, 0)),
        ],
        out_specs=[
            pl.BlockSpec((1, 1, 1, ROW_TILE), lambda b, j: (b, j, 0, 0)),
            pl.BlockSpec((1, 1), lambda b, j: (0, 0)),
        ],
        out_shape=[
            jax.ShapeDtypeStruct((B, HWp // ROW_TILE, 1, ROW_TILE), jnp.int32),
            jax.ShapeDtypeStruct((1, 1), jnp.float32),
        ],
    )(z3, cb_aug)
    idx = idx_out.reshape(B, HWp)[:, :hw].reshape(B * hw)
    return idx, loss_sum[0, 0]


_SC_CORES = 2
_SC_SUBCORES = 16
_NW = _SC_CORES * _SC_SUBCORES


def _make_sc_gather(b_pad):
    b_per_w = b_pad // _NW
    mesh = plsc.VectorSubcoreMesh(core_axis_name="c", subcore_axis_name="s")

    rows_per_tile = NUM_EMB // _SC_SUBCORES

    @functools.partial(
        pl.kernel, mesh=mesh,
        compiler_params=pltpu.CompilerParams(use_tc_tiling_on_sc=False),
        out_type=jax.ShapeDtypeStruct((b_pad, EMB_DIM), jnp.float32),
        scratch_types=[
            pltpu.VMEM((b_per_w,), jnp.int32),
            pltpu.VMEM((b_per_w, EMB_DIM), jnp.float32),
            pltpu.VMEM_SHARED((NUM_EMB, EMB_DIM), jnp.float32),
            pltpu.SemaphoreType.DMA,
        ],
    )
    def sc_gather(table_hbm, idx_hbm, out_hbm, idx_v, rows_v, table_sh, sem):
        s = lax.axis_index("s")
        wid = s * _SC_CORES + lax.axis_index("c")
        base = wid * b_per_w
        pltpu.sync_copy(table_hbm.at[pl.ds(s * rows_per_tile, rows_per_tile)],
                        table_sh.at[pl.ds(s * rows_per_tile, rows_per_tile)])
        pltpu.sync_copy(idx_hbm.at[pl.ds(base, b_per_w)], idx_v)
        plsc.subcore_barrier()
        pltpu.async_copy(table_sh.at[idx_v], rows_v, sem).wait()
        pltpu.sync_copy(rows_v, out_hbm.at[pl.ds(base, b_per_w)])

    return sc_gather


def _conv(x, w, b, stride=1, pad=1):
    o = lax.conv_general_dilated(x, w, (stride, stride), [(pad, pad), (pad, pad)],
                                 dimension_numbers=('NCHW', 'OIHW', 'NCHW'))
    return o + b[None, :, None, None]


def _convT(x, w, b, stride=2, pad=1):
    k = w.shape[2]
    wf = jnp.flip(w, axis=(2, 3)).transpose(1, 0, 2, 3)
    p = k - 1 - pad
    o = lax.conv_general_dilated(x, wf, (1, 1), [(p, p), (p, p)],
                                 lhs_dilation=(stride, stride),
                                 dimension_numbers=('NCHW', 'OIHW', 'NCHW'))
    return o + b[None, :, None, None]


def _res(x, p, name):
    h = jax.nn.relu(_conv(x, p[name + '_w1'], p[name + '_b1']))
    h = _conv(h, p[name + '_w2'], p[name + '_b2'])
    return jax.nn.relu(x + h)


def _encode(x, p):
    x = jax.nn.relu(_conv(x, p['e_d0_w'], p['e_d0_b'], stride=2, pad=1))
    x = _res(x, p, 'e_s0_r0'); x = _res(x, p, 'e_s0_r1')
    x = jax.nn.relu(_conv(x, p['e_d1_w'], p['e_d1_b'], stride=2, pad=1))
    x = _res(x, p, 'e_s1_r0'); x = _res(x, p, 'e_s1_r1')
    return _conv(x, p['e_out_w'], p['e_out_b'])


def _decode(x, p):
    x = _conv(x, p['d_in_w'], p['d_in_b'])
    x = _res(x, p, 'd_s0_r0'); x = _res(x, p, 'd_s0_r1')
    x = jax.nn.relu(_convT(x, p['d_u0_w'], p['d_u0_b']))
    x = _res(x, p, 'd_s1_r0'); x = _res(x, p, 'd_s1_r1')
    return _convT(x, p['d_u1_w'], p['d_u1_b'])


def kernel(x, params):
    z = _encode(x, params)
    B, C, H, W = z.shape
    n = B * H * W
    hw = H * W
    cb = params['codebook']

    eye = jnp.eye(C, dtype=jnp.float32).reshape(C, C, 1, 1)
    z2 = lax.conv_general_dilated(z, eye, (1, 1), [(0, 0), (0, 0)],
                                  dimension_numbers=('NCHW', 'OIHW', 'NCHW'),
                                  precision=jax.lax.Precision.HIGHEST)

    hw_pad = -(-hw // ROW_TILE) * ROW_TILE
    z3 = jnp.pad(z2.reshape(B, C, hw), ((0, 0), (0, 0), (0, hw_pad - hw)))
    idx, loss_sum = _vq_argmin(z3, cb * 2.0, hw)
    loss = 0.25 * loss_sum / (n * C)

    b_pad = -(-n // (8 * _NW)) * (8 * _NW)
    idx_pad = jnp.pad(idx, (0, b_pad - n))
    q = _make_sc_gather(b_pad)(cb, idx_pad)[:n]

    q_img = q.reshape(B, H, W, C).transpose(0, 3, 1, 2)
    q_st = z + (q_img - z)
    recon = _decode(q_st, params)
    return recon, loss
